# baseline (speedup 1.0000x reference)
import functools

import jax
import jax.numpy as jnp
from jax import lax
from jax.experimental import pallas as pl
from jax.experimental.pallas import tpu as pltpu
from jax.experimental.pallas import tpu_sc as plsc

N = 10000
NPAD = 10240
E = 320000
NW = 32
C = 128
NCH = 80
TEDGE = NCH * C
EPAD = NW * TEDGE
ROWS_TOT = NW * NCH
NCH0 = 48
NCH1 = 112
NCHMAX = max(NCH0, NCH1)
NH = 3
F32 = jnp.float32

_SC_PARAMS = pltpu.CompilerParams(
    needs_layout_passes=False, use_tc_tiling_on_sc=False)
_MESH = dict(core_axis_name="c", subcore_axis_name="s", num_cores=2,
             num_subcores=16)
_NSL = NPAD // 16
_SSL = 4 * NPAD // 16


def _edge_logits(asrc, adst, sv, dv, off, valid, eebuf, sidx, k16, h, hs):
    a = plsc.load_gather(asrc, [sv + off])
    b = plsc.load_gather(adst, [dv + off])
    e = a + b
    e = jnp.maximum(e, 0.2 * e)
    ee = jnp.where(valid, jnp.exp(e), 0.0)
    eebuf[hs, pl.ds(k16 * 16, 16)] = ee
    sidx[hs, pl.ds(k16 * 16, 16)] = dv * 4 + h


@functools.lru_cache(maxsize=None)
def _gat_sc_l1():
    @functools.partial(
        pl.kernel,
        out_type=(
            jax.ShapeDtypeStruct((2, NPAD, 48), F32),
            jax.ShapeDtypeStruct((2, 4 * NPAD), F32),
        ),
        mesh=plsc.VectorSubcoreMesh(**_MESH),
        compiler_params=_SC_PARAMS,
        scratch_types=[
            pltpu.VMEM((NH * NPAD,), F32),
            pltpu.VMEM((NH * NPAD,), F32),
            pltpu.VMEM((NCH, C), jnp.int32),
            pltpu.VMEM((NCH, C), jnp.int32),
            pltpu.VMEM((2 * NH, C), jnp.int32),
            pltpu.VMEM((2 * NH, C), F32),
            pltpu.VMEM((C, 48), F32),
            pltpu.VMEM((C, 48), F32),
            pltpu.VMEM_SHARED((NPAD, 48), F32),
            pltpu.VMEM_SHARED((4 * NPAD,), F32),
            pltpu.SemaphoreType.DMA,
            pltpu.SemaphoreType.DMA,
        ],
    )
    def k(hfeat_hbm, avt_hbm, srcr_hbm, dstr_hbm, zacc_hbm, zs_hbm,
          acc_out, s_out,
          asrc, adst, srcv, dstv, sidx, eebuf, rows0, rows1,
          acc_sh, s_sh, gsem0, gsem1):
        cid = lax.axis_index("c")
        sid = lax.axis_index("s")
        wid = sid * 2 + cid
        for h in range(NH):
            pltpu.sync_copy(avt_hbm.at[h], asrc.at[pl.ds(h * NPAD, NPAD)])
            pltpu.sync_copy(avt_hbm.at[NH + h],
                            adst.at[pl.ds(h * NPAD, NPAD)])
        pltpu.sync_copy(srcr_hbm.at[pl.ds(wid * NCH, NCH)], srcv)
        pltpu.sync_copy(dstr_hbm.at[pl.ds(wid * NCH, NCH)], dstv)
        pltpu.sync_copy(zacc_hbm.at[pl.ds(sid * _NSL, _NSL)],
                        acc_sh.at[pl.ds(sid * _NSL, _NSL)])
        pltpu.sync_copy(zs_hbm.at[pl.ds(sid * _SSL, _SSL)],
                        s_sh.at[pl.ds(sid * _SSL, _SSL)])
        plsc.subcore_barrier()

        lane = lax.iota(jnp.int32, 16)
        bufs = ((rows0, gsem0), (rows1, gsem1))

        def issue(j, p):
            pltpu.async_copy(hfeat_hbm.at[srcv.at[j]], bufs[p][0],
                             bufs[p][1])

        def wait_gather(p):
            pltpu.make_async_copy(hfeat_hbm.at[pl.ds(0, C)], bufs[p][0],
                                  bufs[p][1]).wait()

        def compute_ee(j, p):
            base_g = (wid * NCH + j) * C
            for k16 in range(C // 16):
                sv = srcv[j, pl.ds(k16 * 16, 16)]
                dv = dstv[j, pl.ds(k16 * 16, 16)]
                valid = (base_g + k16 * 16 + lane) < E
                for h in range(NH):
                    _edge_logits(asrc, adst, sv, dv, h * NPAD, valid,
                                 eebuf, sidx, k16, h, p * NH + h)

        def mul_scatter(j, p):
            rows = bufs[p][0]

            def mul_body(g, mc):
                ee_vecs = [eebuf[p * NH + h, pl.ds(g * 16, 16)]
                           for h in range(NH)]
                for i16 in range(16):
                    i = g * 16 + i16
                    for h in range(NH):
                        ee_v = jnp.full((16,), ee_vecs[h][i16], F32)
                        rows[i, pl.ds(h * 16, 16)] = (
                            rows[i, pl.ds(h * 16, 16)] * ee_v)
                return mc
            lax.fori_loop(0, C // 16, mul_body, 0)

            pltpu.sync_copy(rows, acc_sh.at[dstv.at[j]], add=True)
            for h in range(NH):
                pltpu.sync_copy(eebuf.at[p * NH + h],
                                s_sh.at[sidx.at[p * NH + h]], add=True)

        issue(0, 0)

        def pair_body(j2, carry):
            j0 = j2 * 2
            issue(j0 + 1, 1)
            compute_ee(j0, 0)
            wait_gather(0)
            mul_scatter(j0, 0)

            @pl.when(j2 < NCH // 2 - 1)
            def _():
                issue(j0 + 2, 0)
            compute_ee(j0 + 1, 1)
            wait_gather(1)
            mul_scatter(j0 + 1, 1)
            return carry
        lax.fori_loop(0, NCH // 2, pair_body, 0)
        plsc.subcore_barrier()
        pltpu.sync_copy(acc_sh.at[pl.ds(sid * _NSL, _NSL)],
                        acc_out.at[cid, pl.ds(sid * _NSL, _NSL)])
        pltpu.sync_copy(s_sh.at[pl.ds(sid * _SSL, _SSL)],
                        s_out.at[cid, pl.ds(sid * _SSL, _SSL)])

    return k


@functools.lru_cache(maxsize=None)
def _gat_sc_l2():
    @functools.partial(
        pl.kernel,
        out_type=(
            jax.ShapeDtypeStruct((2, NH, NPAD, 64), F32),
            jax.ShapeDtypeStruct((2, 4 * NPAD), F32),
        ),
        mesh=plsc.VectorSubcoreMesh(**_MESH),
        compiler_params=_SC_PARAMS,
        scratch_types=[
            pltpu.VMEM((NPAD,), F32),
            pltpu.VMEM((NPAD,), F32),
            pltpu.VMEM((NCHMAX, C), jnp.int32),
            pltpu.VMEM((NCHMAX, C), jnp.int32),
            pltpu.VMEM((2, C), jnp.int32),
            pltpu.VMEM((2, C), F32),
            pltpu.VMEM((C, 64), F32),
            pltpu.VMEM((C, 64), F32),
            pltpu.VMEM_SHARED((NPAD, 64), F32),
            pltpu.VMEM_SHARED((4 * NPAD,), F32),
            pltpu.SemaphoreType.DMA,
            pltpu.SemaphoreType.DMA,
        ],
    )
    def k(hf0_hbm, hf1_hbm, hf2_hbm, avt_hbm, srcr_hbm, dstr_hbm,
          zacc_hbm, zs_hbm,
          acc_out, s_out,
          asrc, adst, srcv, dstv, sidx, eebuf, rows0, rows1,
          acc_sh, s_sh, gsem0, gsem1):
        cid = lax.axis_index("c")
        sid = lax.axis_index("s")
        hf_hbm = (hf0_hbm, hf1_hbm, hf2_hbm)
        rowbase = jnp.where(cid == 0, sid * NCH0, 16 * NCH0 + sid * NCH1)
        npairs = jnp.where(cid == 0, NCH0 // 2, NCH1 // 2)

        @pl.when(cid == 0)
        def _():
            pltpu.sync_copy(srcr_hbm.at[pl.ds(rowbase, NCH0)],
                            srcv.at[pl.ds(0, NCH0)])
            pltpu.sync_copy(dstr_hbm.at[pl.ds(rowbase, NCH0)],
                            dstv.at[pl.ds(0, NCH0)])

        @pl.when(cid == 1)
        def _():
            pltpu.sync_copy(srcr_hbm.at[pl.ds(rowbase, NCH1)],
                            srcv.at[pl.ds(0, NCH1)])
            pltpu.sync_copy(dstr_hbm.at[pl.ds(rowbase, NCH1)],
                            dstv.at[pl.ds(0, NCH1)])
        pltpu.sync_copy(zs_hbm.at[pl.ds(sid * _SSL, _SSL)],
                        s_sh.at[pl.ds(sid * _SSL, _SSL)])

        lane = lax.iota(jnp.int32, 16)
        bufs = ((rows0, gsem0), (rows1, gsem1))

        for h in range(NH):
            pltpu.sync_copy(avt_hbm.at[h], asrc)
            pltpu.sync_copy(avt_hbm.at[NH + h], adst)
            pltpu.sync_copy(zacc_hbm.at[pl.ds(sid * _NSL, _NSL)],
                            acc_sh.at[pl.ds(sid * _NSL, _NSL)])
            plsc.subcore_barrier()

            def issue(j, p):
                pltpu.async_copy(hf_hbm[h].at[srcv.at[j]], bufs[p][0],
                                 bufs[p][1])

            def wait_gather(p):
                pltpu.make_async_copy(hf_hbm[h].at[pl.ds(0, C)],
                                      bufs[p][0], bufs[p][1]).wait()

            def compute_ee(j, p):
                base_g = (rowbase + j) * C
                for k16 in range(C // 16):
                    sv = srcv[j, pl.ds(k16 * 16, 16)]
                    dv = dstv[j, pl.ds(k16 * 16, 16)]
                    valid = (base_g + k16 * 16 + lane) < E
                    _edge_logits(asrc, adst, sv, dv, 0, valid,
                                 eebuf, sidx, k16, h, p)

            def mul_scatter(j, p):
                rows = bufs[p][0]

                def mul_body(g, mc):
                    ee_vec = eebuf[p, pl.ds(g * 16, 16)]
                    for i16 in range(16):
                        i = g * 16 + i16
                        ee_v = jnp.full((16,), ee_vec[i16], F32)
                        for b in range(4):
                            rows[i, pl.ds(b * 16, 16)] = (
                                rows[i, pl.ds(b * 16, 16)] * ee_v)
                    return mc
                lax.fori_loop(0, C // 16, mul_body, 0)

                pltpu.sync_copy(rows, acc_sh.at[dstv.at[j]], add=True)
                pltpu.sync_copy(eebuf.at[p], s_sh.at[sidx.at[p]], add=True)

            issue(0, 0)

            def pair_body(j2, carry):
                j0 = j2 * 2
                issue(j0 + 1, 1)
                compute_ee(j0, 0)
                wait_gather(0)
                mul_scatter(j0, 0)

                @pl.when(j2 < npairs - 1)
                def _():
                    issue(j0 + 2, 0)
                compute_ee(j0 + 1, 1)
                wait_gather(1)
                mul_scatter(j0 + 1, 1)
                return carry
            lax.fori_loop(0, npairs, pair_body, 0)
            plsc.subcore_barrier()
            pltpu.sync_copy(acc_sh.at[pl.ds(sid * _NSL, _NSL)],
                            acc_out.at[cid, h, pl.ds(sid * _NSL, _NSL)])
            plsc.subcore_barrier()
        pltpu.sync_copy(s_sh.at[pl.ds(sid * _SSL, _SSL)],
                        s_out.at[cid, pl.ds(sid * _SSL, _SSL)])

    return k


def _dot(a, b):
    return lax.dot_general(a, b, (((1,), (0,)), ((), ())),
                           preferred_element_type=F32)


def _dot_rt(a, b):
    return lax.dot_general(a, b, (((1,), (1,)), ((), ())),
                           preferred_element_type=F32)


def _tc1_body(xp, w1cat, a1t, hfeat_out, avt_out):
    h = _dot(xp[...], w1cat[...])
    hfeat_out[...] = h
    avt_out[...] = _dot_rt(a1t[...], h)


_tc1 = pl.pallas_call(
    _tc1_body,
    out_shape=(jax.ShapeDtypeStruct((NPAD, 48), F32),
               jax.ShapeDtypeStruct((8, NPAD), F32)))


def _elu(x):
    return jnp.where(x > 0, x, jnp.exp(jnp.minimum(x, 0.0)) - 1.0)


def _tc2_body(acc1, s1, w2cat, a2t, e1, hf0_out, hf1_out, hf2_out, avt_out):
    accsum = acc1[0] + acc1[1]
    ssum = s1[0] + s1[1]
    rmat = _dot(1.0 / (ssum + 1e-16), e1[...])
    h1 = _elu(accsum * rmat)
    h2f = _dot(h1, w2cat[...])
    hf0_out[...] = h2f[:, 0:64]
    hf1_out[...] = h2f[:, 64:128]
    hf2_out[...] = h2f[:, 128:192]
    avt_out[...] = _dot_rt(a2t[...], h2f)


_tc2 = pl.pallas_call(
    _tc2_body,
    out_shape=(jax.ShapeDtypeStruct((NPAD, 64), F32),
               jax.ShapeDtypeStruct((NPAD, 64), F32),
               jax.ShapeDtypeStruct((NPAD, 64), F32),
               jax.ShapeDtypeStruct((8, NPAD), F32)))


def _tc3_body(acc2, s2, e2, att_w2, wd1, bd1r, wd2, bd2r, out):
    accsum = acc2[0] + acc2[1]
    acat = jnp.concatenate(
        [accsum[0], accsum[1], accsum[2]], axis=1)
    ssum = s2[0] + s2[1]
    rmat = _dot(1.0 / (ssum + 1e-16), e2[...])
    h2 = _elu(acat * rmat)
    th = jnp.tanh(_dot(h2, att_w2[...]))
    ridx = lax.broadcasted_iota(jnp.int32, (NPAD, 1), 0)
    z = jnp.where(ridx < N, th, -1e30)
    p = jnp.exp(z - jnp.max(z))
    scores = p / jnp.sum(p)
    w = h2 * scores
    d1 = jnp.maximum(_dot(w, wd1[...]) + bd1r[...], 0.0)
    out[...] = _dot(d1, wd2[...]) + bd2r[...]


_tc3 = pl.pallas_call(
    _tc3_body,
    out_shape=jax.ShapeDtypeStruct((NPAD, 1), F32),
    compiler_params=pltpu.CompilerParams(
        vmem_limit_bytes=100 * 1024 * 1024))


def kernel(x, edge_index, W1, a_src1, a_dst1, W2, a_src2, a_dst2, att_w,
           Wd1, bd1, Wd2, bd2):
    xp = jnp.pad(x.astype(F32), ((0, NPAD - N), (0, 0)))
    src = jnp.pad(edge_index[0].astype(jnp.int32), (0, EPAD - E))
    dst = jnp.pad(edge_index[1].astype(jnp.int32), (0, EPAD - E))
    srcp = src.reshape(ROWS_TOT, C)
    dstp = dst.reshape(ROWS_TOT, C)

    w1cat = W1.transpose(1, 0, 2).reshape(11, 48)
    w2cat = W2.transpose(1, 0, 2).reshape(48, 192)
    a1t = jnp.zeros((8, 48), F32)
    a2t = jnp.zeros((8, 192), F32)
    e1 = jnp.zeros((4, 48), F32)
    e2 = jnp.zeros((4, 192), F32)
    for h in range(NH):
        a1t = a1t.at[h, h * 16:(h + 1) * 16].set(a_src1[h])
        a1t = a1t.at[NH + h, h * 16:(h + 1) * 16].set(a_dst1[h])
        a2t = a2t.at[h, h * 64:(h + 1) * 64].set(a_src2[h])
        a2t = a2t.at[NH + h, h * 64:(h + 1) * 64].set(a_dst2[h])
        e1 = e1.at[h, h * 16:(h + 1) * 16].set(1.0)
        e2 = e2.at[h, h * 64:(h + 1) * 64].set(1.0)

    zacc1 = jnp.zeros((NPAD, 48), F32)
    zacc2 = jnp.zeros((NPAD, 64), F32)
    zs = jnp.zeros((4 * NPAD,), F32)

    hfeat1, avt1 = _tc1(xp, w1cat, a1t)
    acc1, s1 = _gat_sc_l1()(hfeat1, avt1, srcp, dstp, zacc1, zs)
    hf0, hf1, hf2, avt2 = _tc2(acc1, s1.reshape(2, NPAD, 4), w2cat, a2t, e1)
    acc2, s2 = _gat_sc_l2()(hf0, hf1, hf2, avt2, srcp, dstp, zacc2, zs)
    o = _tc3(acc2, s2.reshape(2, NPAD, 4), e2, att_w.reshape(192, 1),
             Wd1, bd1.reshape(1, 128), Wd2, bd2.reshape(1, 1))
    return o[:N, 0]


# speedup vs baseline: 2.3694x; 2.3694x over previous
import functools

import jax
import jax.numpy as jnp
from jax import lax
from jax.experimental import pallas as pl
from jax.experimental.pallas import tpu as pltpu
from jax.experimental.pallas import tpu_sc as plsc

N = 10000
NPAD = 10240
E = 320000
NW = 32
C = 128
NCH = 80
TEDGE = NCH * C
EPAD = NW * TEDGE
ROWS_TOT = NW * NCH
NCH0 = 80
NCH1 = 80
NCHMAX = max(NCH0, NCH1)
NH = 3
F32 = jnp.float32

_SC_PARAMS = pltpu.CompilerParams(
    needs_layout_passes=False, use_tc_tiling_on_sc=False)
_MESH = dict(core_axis_name="c", subcore_axis_name="s", num_cores=2,
             num_subcores=16)
_NSL = NPAD // 16
_SSL = 4 * NPAD // 16


def _edge_logits(asrc, adst, sv, dv, off, valid, eebuf, sidx, k16, h, hs):
    a = plsc.load_gather(asrc, [sv + off])
    b = plsc.load_gather(adst, [dv + off])
    e = a + b
    e = jnp.maximum(e, 0.2 * e)
    ee = jnp.where(valid, jnp.exp(e), 0.0)
    eebuf[hs, pl.ds(k16 * 16, 16)] = ee
    sidx[hs, pl.ds(k16 * 16, 16)] = dv * 4 + h


@functools.lru_cache(maxsize=None)
def _gat_sc_l1():
    @functools.partial(
        pl.kernel,
        out_type=(
            jax.ShapeDtypeStruct((2, NPAD, 48), F32),
            jax.ShapeDtypeStruct((2, 4 * NPAD), F32),
        ),
        mesh=plsc.VectorSubcoreMesh(**_MESH),
        compiler_params=_SC_PARAMS,
        scratch_types=[
            pltpu.VMEM((NH * NPAD,), F32),
            pltpu.VMEM((NH * NPAD,), F32),
            pltpu.VMEM((NCH, C), jnp.int32),
            pltpu.VMEM((NCH, C), jnp.int32),
            pltpu.VMEM((2 * NH, C), jnp.int32),
            pltpu.VMEM((2 * NH, C), F32),
            pltpu.VMEM((C, 48), F32),
            pltpu.VMEM((C, 48), F32),
            pltpu.VMEM_SHARED((NPAD, 48), F32),
            pltpu.VMEM_SHARED((4 * NPAD,), F32),
            pltpu.SemaphoreType.DMA,
            pltpu.SemaphoreType.DMA,
        ],
    )
    def k(hfeat_hbm, avt_hbm, srcr_hbm, dstr_hbm, zacc_hbm, zs_hbm,
          acc_out, s_out,
          asrc, adst, srcv, dstv, sidx, eebuf, rows0, rows1,
          acc_sh, s_sh, gsem0, gsem1):
        cid = lax.axis_index("c")
        sid = lax.axis_index("s")
        wid = sid * 2 + cid
        for h in range(NH):
            pltpu.sync_copy(avt_hbm.at[h], asrc.at[pl.ds(h * NPAD, NPAD)])
            pltpu.sync_copy(avt_hbm.at[NH + h],
                            adst.at[pl.ds(h * NPAD, NPAD)])
        pltpu.sync_copy(srcr_hbm.at[pl.ds(wid * NCH, NCH)], srcv)
        pltpu.sync_copy(dstr_hbm.at[pl.ds(wid * NCH, NCH)], dstv)
        pltpu.sync_copy(zacc_hbm.at[pl.ds(sid * _NSL, _NSL)],
                        acc_sh.at[pl.ds(sid * _NSL, _NSL)])
        pltpu.sync_copy(zs_hbm.at[pl.ds(sid * _SSL, _SSL)],
                        s_sh.at[pl.ds(sid * _SSL, _SSL)])
        plsc.subcore_barrier()

        lane = lax.iota(jnp.int32, 16)
        bufs = ((rows0, gsem0), (rows1, gsem1))

        def issue(j, p):
            pltpu.async_copy(hfeat_hbm.at[srcv.at[j]], bufs[p][0],
                             bufs[p][1])

        def wait_gather(p):
            pltpu.make_async_copy(hfeat_hbm.at[pl.ds(0, C)], bufs[p][0],
                                  bufs[p][1]).wait()

        def compute_ee(j, p):
            base_g = (wid * NCH + j) * C
            for k16 in range(C // 16):
                sv = srcv[j, pl.ds(k16 * 16, 16)]
                dv = dstv[j, pl.ds(k16 * 16, 16)]
                valid = (base_g + k16 * 16 + lane) < E
                for h in range(NH):
                    _edge_logits(asrc, adst, sv, dv, h * NPAD, valid,
                                 eebuf, sidx, k16, h, p * NH + h)

        def mul_scatter(j, p):
            rows = bufs[p][0]

            def mul_body(g, mc):
                ee_vecs = [eebuf[p * NH + h, pl.ds(g * 16, 16)]
                           for h in range(NH)]
                for i16 in range(16):
                    i = g * 16 + i16
                    for h in range(NH):
                        ee_v = jnp.full((16,), ee_vecs[h][i16], F32)
                        rows[i, pl.ds(h * 16, 16)] = (
                            rows[i, pl.ds(h * 16, 16)] * ee_v)
                return mc
            lax.fori_loop(0, C // 16, mul_body, 0)

            pltpu.sync_copy(rows, acc_sh.at[dstv.at[j]], add=True)
            for h in range(NH):
                pltpu.sync_copy(eebuf.at[p * NH + h],
                                s_sh.at[sidx.at[p * NH + h]], add=True)

        issue(0, 0)

        def pair_body(j2, carry):
            j0 = j2 * 2
            issue(j0 + 1, 1)
            compute_ee(j0, 0)
            wait_gather(0)
            mul_scatter(j0, 0)

            @pl.when(j2 < NCH // 2 - 1)
            def _():
                issue(j0 + 2, 0)
            compute_ee(j0 + 1, 1)
            wait_gather(1)
            mul_scatter(j0 + 1, 1)
            return carry
        lax.fori_loop(0, NCH // 2, pair_body, 0)
        plsc.subcore_barrier()
        pltpu.sync_copy(acc_sh.at[pl.ds(sid * _NSL, _NSL)],
                        acc_out.at[cid, pl.ds(sid * _NSL, _NSL)])
        pltpu.sync_copy(s_sh.at[pl.ds(sid * _SSL, _SSL)],
                        s_out.at[cid, pl.ds(sid * _SSL, _SSL)])

    return k


@functools.lru_cache(maxsize=None)
def _gat_sc_l2():
    @functools.partial(
        pl.kernel,
        out_type=(
            jax.ShapeDtypeStruct((2, NH, NPAD, 64), F32),
            jax.ShapeDtypeStruct((2, 4 * NPAD), F32),
        ),
        mesh=plsc.VectorSubcoreMesh(**_MESH),
        compiler_params=_SC_PARAMS,
        scratch_types=[
            pltpu.VMEM((NPAD,), F32),
            pltpu.VMEM((NPAD,), F32),
            pltpu.VMEM((NCHMAX, C), jnp.int32),
            pltpu.VMEM((NCHMAX, C), jnp.int32),
            pltpu.VMEM((2, C), jnp.int32),
            pltpu.VMEM((2, C), F32),
            pltpu.VMEM((C, 64), F32),
            pltpu.VMEM((C, 64), F32),
            pltpu.VMEM_SHARED((NPAD, 64), F32),
            pltpu.VMEM_SHARED((4 * NPAD,), F32),
            pltpu.SemaphoreType.DMA,
            pltpu.SemaphoreType.DMA,
        ],
    )
    def k(hf0_hbm, hf1_hbm, hf2_hbm, avt_hbm, srcr_hbm, dstr_hbm,
          zacc_hbm, zs_hbm,
          acc_out, s_out,
          asrc, adst, srcv, dstv, sidx, eebuf, rows0, rows1,
          acc_sh, s_sh, gsem0, gsem1):
        cid = lax.axis_index("c")
        sid = lax.axis_index("s")
        hf_hbm = (hf0_hbm, hf1_hbm, hf2_hbm)
        rowbase = jnp.where(cid == 0, sid * NCH0, 16 * NCH0 + sid * NCH1)
        npairs = jnp.where(cid == 0, NCH0 // 2, NCH1 // 2)

        @pl.when(cid == 0)
        def _():
            pltpu.sync_copy(srcr_hbm.at[pl.ds(rowbase, NCH0)],
                            srcv.at[pl.ds(0, NCH0)])
            pltpu.sync_copy(dstr_hbm.at[pl.ds(rowbase, NCH0)],
                            dstv.at[pl.ds(0, NCH0)])

        @pl.when(cid == 1)
        def _():
            pltpu.sync_copy(srcr_hbm.at[pl.ds(rowbase, NCH1)],
                            srcv.at[pl.ds(0, NCH1)])
            pltpu.sync_copy(dstr_hbm.at[pl.ds(rowbase, NCH1)],
                            dstv.at[pl.ds(0, NCH1)])
        pltpu.sync_copy(zs_hbm.at[pl.ds(sid * _SSL, _SSL)],
                        s_sh.at[pl.ds(sid * _SSL, _SSL)])

        lane = lax.iota(jnp.int32, 16)
        bufs = ((rows0, gsem0), (rows1, gsem1))

        for h in range(NH):
            pltpu.sync_copy(avt_hbm.at[h], asrc)
            pltpu.sync_copy(avt_hbm.at[NH + h], adst)
            pltpu.sync_copy(zacc_hbm.at[pl.ds(sid * _NSL, _NSL)],
                            acc_sh.at[pl.ds(sid * _NSL, _NSL)])
            plsc.subcore_barrier()

            def issue(j, p):
                pltpu.async_copy(hf_hbm[h].at[srcv.at[j]], bufs[p][0],
                                 bufs[p][1])

            def wait_gather(p):
                pltpu.make_async_copy(hf_hbm[h].at[pl.ds(0, C)],
                                      bufs[p][0], bufs[p][1]).wait()

            def compute_ee(j, p):
                base_g = (rowbase + j) * C
                for k16 in range(C // 16):
                    sv = srcv[j, pl.ds(k16 * 16, 16)]
                    dv = dstv[j, pl.ds(k16 * 16, 16)]
                    valid = (base_g + k16 * 16 + lane) < E
                    _edge_logits(asrc, adst, sv, dv, 0, valid,
                                 eebuf, sidx, k16, h, p)

            def mul_scatter(j, p):
                rows = bufs[p][0]

                def mul_body(g, mc):
                    ee_vec = eebuf[p, pl.ds(g * 16, 16)]
                    for i16 in range(16):
                        i = g * 16 + i16
                        ee_v = jnp.full((16,), ee_vec[i16], F32)
                        for b in range(4):
                            rows[i, pl.ds(b * 16, 16)] = (
                                rows[i, pl.ds(b * 16, 16)] * ee_v)
                    return mc
                lax.fori_loop(0, C // 16, mul_body, 0)

                pltpu.sync_copy(rows, acc_sh.at[dstv.at[j]], add=True)
                pltpu.sync_copy(eebuf.at[p], s_sh.at[sidx.at[p]], add=True)

            issue(0, 0)

            def pair_body(j2, carry):
                j0 = j2 * 2
                issue(j0 + 1, 1)
                compute_ee(j0, 0)
                wait_gather(0)
                mul_scatter(j0, 0)

                @pl.when(j2 < npairs - 1)
                def _():
                    issue(j0 + 2, 0)
                compute_ee(j0 + 1, 1)
                wait_gather(1)
                mul_scatter(j0 + 1, 1)
                return carry
            lax.fori_loop(0, npairs, pair_body, 0)
            plsc.subcore_barrier()
            pltpu.sync_copy(acc_sh.at[pl.ds(sid * _NSL, _NSL)],
                            acc_out.at[cid, h, pl.ds(sid * _NSL, _NSL)])
            plsc.subcore_barrier()
        pltpu.sync_copy(s_sh.at[pl.ds(sid * _SSL, _SSL)],
                        s_out.at[cid, pl.ds(sid * _SSL, _SSL)])

    return k


def _dot(a, b):
    return lax.dot_general(a, b, (((1,), (0,)), ((), ())),
                           preferred_element_type=F32)


def _dot_rt(a, b):
    return lax.dot_general(a, b, (((1,), (1,)), ((), ())),
                           preferred_element_type=F32)


def _tc1_body(xp, w1cat, a1t, hfeat_out, avt_out):
    h = _dot(xp[...], w1cat[...])
    hfeat_out[...] = h
    avt_out[...] = _dot_rt(a1t[...], h)


_tc1 = pl.pallas_call(
    _tc1_body,
    out_shape=(jax.ShapeDtypeStruct((NPAD, 48), F32),
               jax.ShapeDtypeStruct((8, NPAD), F32)))


def _elu(x):
    return jnp.where(x > 0, x, jnp.exp(jnp.minimum(x, 0.0)) - 1.0)


def _tc2_body(acc1, s1, w2cat, a2t, e1, hf0_out, hf1_out, hf2_out, avt_out):
    accsum = acc1[0] + acc1[1]
    ssum = s1[0] + s1[1]
    rmat = _dot(1.0 / (ssum + 1e-16), e1[...])
    h1 = _elu(accsum * rmat)
    h2f = _dot(h1, w2cat[...])
    hf0_out[...] = h2f[:, 0:64]
    hf1_out[...] = h2f[:, 64:128]
    hf2_out[...] = h2f[:, 128:192]
    avt_out[...] = _dot_rt(a2t[...], h2f)


_tc2 = pl.pallas_call(
    _tc2_body,
    out_shape=(jax.ShapeDtypeStruct((NPAD, 64), F32),
               jax.ShapeDtypeStruct((NPAD, 64), F32),
               jax.ShapeDtypeStruct((NPAD, 64), F32),
               jax.ShapeDtypeStruct((8, NPAD), F32)))


def _tc3_body(acc2, s2, e2, att_w2, wd1, bd1r, wd2, bd2r, out):
    accsum = acc2[0] + acc2[1]
    acat = jnp.concatenate(
        [accsum[0], accsum[1], accsum[2]], axis=1)
    ssum = s2[0] + s2[1]
    rmat = _dot(1.0 / (ssum + 1e-16), e2[...])
    h2 = _elu(acat * rmat)
    th = jnp.tanh(_dot(h2, att_w2[...]))
    ridx = lax.broadcasted_iota(jnp.int32, (NPAD, 1), 0)
    z = jnp.where(ridx < N, th, -1e30)
    p = jnp.exp(z - jnp.max(z))
    scores = p / jnp.sum(p)
    w = h2 * scores
    d1 = jnp.maximum(_dot(w, wd1[...]) + bd1r[...], 0.0)
    out[...] = _dot(d1, wd2[...]) + bd2r[...]


_tc3 = pl.pallas_call(
    _tc3_body,
    out_shape=jax.ShapeDtypeStruct((NPAD, 1), F32),
    compiler_params=pltpu.CompilerParams(
        vmem_limit_bytes=100 * 1024 * 1024))


def kernel(x, edge_index, W1, a_src1, a_dst1, W2, a_src2, a_dst2, att_w,
           Wd1, bd1, Wd2, bd2):
    xp = jnp.pad(x.astype(F32), ((0, NPAD - N), (0, 0)))
    fill = jnp.arange(EPAD - E, dtype=jnp.int32) % N
    src = jnp.concatenate([edge_index[0].astype(jnp.int32), fill])
    dst = jnp.concatenate([edge_index[1].astype(jnp.int32), fill])
    srcp = src.reshape(ROWS_TOT, C)
    dstp = dst.reshape(ROWS_TOT, C)

    w1cat = W1.transpose(1, 0, 2).reshape(11, 48)
    w2cat = W2.transpose(1, 0, 2).reshape(48, 192)
    a1t = jnp.zeros((8, 48), F32)
    a2t = jnp.zeros((8, 192), F32)
    e1 = jnp.zeros((4, 48), F32)
    e2 = jnp.zeros((4, 192), F32)
    for h in range(NH):
        a1t = a1t.at[h, h * 16:(h + 1) * 16].set(a_src1[h])
        a1t = a1t.at[NH + h, h * 16:(h + 1) * 16].set(a_dst1[h])
        a2t = a2t.at[h, h * 64:(h + 1) * 64].set(a_src2[h])
        a2t = a2t.at[NH + h, h * 64:(h + 1) * 64].set(a_dst2[h])
        e1 = e1.at[h, h * 16:(h + 1) * 16].set(1.0)
        e2 = e2.at[h, h * 64:(h + 1) * 64].set(1.0)

    zacc1 = jnp.zeros((NPAD, 48), F32)
    zacc2 = jnp.zeros((NPAD, 64), F32)
    zs = jnp.zeros((4 * NPAD,), F32)

    hfeat1, avt1 = _tc1(xp, w1cat, a1t)
    acc1, s1 = _gat_sc_l1()(hfeat1, avt1, srcp, dstp, zacc1, zs)
    hf0, hf1, hf2, avt2 = _tc2(acc1, s1.reshape(2, NPAD, 4), w2cat, a2t, e1)
    acc2, s2 = _gat_sc_l2()(hf0, hf1, hf2, avt2, srcp, dstp, zacc2, zs)
    o = _tc3(acc2, s2.reshape(2, NPAD, 4), e2, att_w.reshape(192, 1),
             Wd1, bd1.reshape(1, 128), Wd2, bd2.reshape(1, 1))
    return o[:N, 0]


# speedup vs baseline: 2.7337x; 1.1538x over previous
import functools

import jax
import jax.numpy as jnp
from jax import lax
from jax.experimental import pallas as pl
from jax.experimental.pallas import tpu as pltpu
from jax.experimental.pallas import tpu_sc as plsc

N = 10000
NPAD = 10240
E = 320000
NW = 32
C = 128
NCH = 80
TEDGE = NCH * C
EPAD = NW * TEDGE
ROWS_TOT = NW * NCH
NCH0 = 80
NCH1 = 80
NCHMAX = max(NCH0, NCH1)
NH = 3
F32 = jnp.float32

_SC_PARAMS = pltpu.CompilerParams(
    needs_layout_passes=False, use_tc_tiling_on_sc=False)
_MESH = dict(core_axis_name="c", subcore_axis_name="s", num_cores=2,
             num_subcores=16)
_NSL = NPAD // 16
_SSL = 4 * NPAD // 16


def _edge_logits(asrc, adst, sv, dv, off, valid, eebuf, sidx, k16, h, hs):
    a = plsc.load_gather(asrc, [sv + off])
    b = plsc.load_gather(adst, [dv + off])
    e = a + b
    e = jnp.maximum(e, 0.2 * e)
    ee = jnp.where(valid, jnp.exp(e), 0.0)
    eebuf[hs, pl.ds(k16 * 16, 16)] = ee
    sidx[hs, pl.ds(k16 * 16, 16)] = dv * 4 + h


@functools.lru_cache(maxsize=None)
def _gat_sc_l1():
    @functools.partial(
        pl.kernel,
        out_type=(
            jax.ShapeDtypeStruct((2, NPAD, 48), F32),
            jax.ShapeDtypeStruct((2, 4 * NPAD), F32),
        ),
        mesh=plsc.VectorSubcoreMesh(**_MESH),
        compiler_params=_SC_PARAMS,
        scratch_types=[
            pltpu.VMEM((NH * NPAD,), F32),
            pltpu.VMEM((NH * NPAD,), F32),
            pltpu.VMEM((NCH, C), jnp.int32),
            pltpu.VMEM((NCH, C), jnp.int32),
            pltpu.VMEM((2 * NH, C), jnp.int32),
            pltpu.VMEM((2 * NH, C), F32),
            pltpu.VMEM((C, 48), F32),
            pltpu.VMEM((C, 48), F32),
            pltpu.VMEM_SHARED((NPAD, 48), F32),
            pltpu.VMEM_SHARED((4 * NPAD,), F32),
            pltpu.SemaphoreType.DMA,
            pltpu.SemaphoreType.DMA,
            pltpu.SemaphoreType.DMA,
            pltpu.SemaphoreType.DMA,
        ],
    )
    def k(hfeat_hbm, avt_hbm, srcr_hbm, dstr_hbm, zacc_hbm, zs_hbm,
          acc_out, s_out,
          asrc, adst, srcv, dstv, sidx, eebuf, rows0, rows1,
          acc_sh, s_sh, gsem0, gsem1, ssem0, ssem1):
        cid = lax.axis_index("c")
        sid = lax.axis_index("s")
        wid = sid * 2 + cid
        for h in range(NH):
            pltpu.sync_copy(avt_hbm.at[h], asrc.at[pl.ds(h * NPAD, NPAD)])
            pltpu.sync_copy(avt_hbm.at[NH + h],
                            adst.at[pl.ds(h * NPAD, NPAD)])
        pltpu.sync_copy(srcr_hbm.at[pl.ds(wid * NCH, NCH)], srcv)
        pltpu.sync_copy(dstr_hbm.at[pl.ds(wid * NCH, NCH)], dstv)
        pltpu.sync_copy(zacc_hbm.at[pl.ds(sid * _NSL, _NSL)],
                        acc_sh.at[pl.ds(sid * _NSL, _NSL)])
        pltpu.sync_copy(zs_hbm.at[pl.ds(sid * _SSL, _SSL)],
                        s_sh.at[pl.ds(sid * _SSL, _SSL)])
        plsc.subcore_barrier()

        lane = lax.iota(jnp.int32, 16)
        bufs = ((rows0, gsem0, ssem0), (rows1, gsem1, ssem1))

        def issue(j, p):
            pltpu.async_copy(hfeat_hbm.at[srcv.at[j]], bufs[p][0],
                             bufs[p][1])

        def wait_gather(p):
            pltpu.make_async_copy(hfeat_hbm.at[pl.ds(0, C)], bufs[p][0],
                                  bufs[p][1]).wait()

        def issue_scatters(j, p):
            pltpu.async_copy(bufs[p][0], acc_sh.at[dstv.at[j]],
                             bufs[p][2], add=True)
            for h in range(NH):
                pltpu.async_copy(eebuf.at[p * NH + h],
                                 s_sh.at[sidx.at[p * NH + h]],
                                 bufs[p][2], add=True)

        def wait_scatters(p):
            pltpu.make_async_copy(hfeat_hbm.at[pl.ds(0, C)], bufs[p][0],
                                  bufs[p][2]).wait()
            for h in range(NH):
                pltpu.make_async_copy(avt_hbm.at[h, pl.ds(0, C)],
                                      eebuf.at[p * NH + h],
                                      bufs[p][2]).wait()

        def compute_ee(j, p):
            base_g = (wid * NCH + j) * C
            for k16 in range(C // 16):
                sv = srcv[j, pl.ds(k16 * 16, 16)]
                dv = dstv[j, pl.ds(k16 * 16, 16)]
                valid = (base_g + k16 * 16 + lane) < E
                for h in range(NH):
                    _edge_logits(asrc, adst, sv, dv, h * NPAD, valid,
                                 eebuf, sidx, k16, h, p * NH + h)

        def mul(p):
            rows = bufs[p][0]

            def mul_body(g, mc):
                ee_vecs = [eebuf[p * NH + h, pl.ds(g * 16, 16)]
                           for h in range(NH)]
                for i16 in range(16):
                    i = g * 16 + i16
                    for h in range(NH):
                        ee_v = jnp.full((16,), ee_vecs[h][i16], F32)
                        rows[i, pl.ds(h * 16, 16)] = (
                            rows[i, pl.ds(h * 16, 16)] * ee_v)
                return mc
            lax.fori_loop(0, C // 16, mul_body, 0)

        issue(0, 0)
        issue(1, 1)

        def pair_body(j2, carry):
            j0 = j2 * 2
            compute_ee(j0, 0)

            @pl.when(j2 > 0)
            def _():
                wait_scatters(1)
                issue(j0 + 1, 1)
            wait_gather(0)
            mul(0)
            issue_scatters(j0, 0)
            compute_ee(j0 + 1, 1)
            wait_scatters(0)

            @pl.when(j0 + 2 < NCH)
            def _():
                issue(j0 + 2, 0)
            wait_gather(1)
            mul(1)
            issue_scatters(j0 + 1, 1)
            return carry
        lax.fori_loop(0, NCH // 2, pair_body, 0)
        wait_scatters(1)
        plsc.subcore_barrier()
        pltpu.sync_copy(acc_sh.at[pl.ds(sid * _NSL, _NSL)],
                        acc_out.at[cid, pl.ds(sid * _NSL, _NSL)])
        pltpu.sync_copy(s_sh.at[pl.ds(sid * _SSL, _SSL)],
                        s_out.at[cid, pl.ds(sid * _SSL, _SSL)])

    return k


@functools.lru_cache(maxsize=None)
def _gat_sc_l2():
    @functools.partial(
        pl.kernel,
        out_type=(
            jax.ShapeDtypeStruct((2, NH, NPAD, 64), F32),
            jax.ShapeDtypeStruct((2, 4 * NPAD), F32),
        ),
        mesh=plsc.VectorSubcoreMesh(**_MESH),
        compiler_params=_SC_PARAMS,
        scratch_types=[
            pltpu.VMEM((NPAD,), F32),
            pltpu.VMEM((NPAD,), F32),
            pltpu.VMEM((NCHMAX, C), jnp.int32),
            pltpu.VMEM((NCHMAX, C), jnp.int32),
            pltpu.VMEM((4, C), jnp.int32),
            pltpu.VMEM((4, C), F32),
            pltpu.VMEM((C, 64), F32),
            pltpu.VMEM((C, 64), F32),
            pltpu.VMEM((C, 64), F32),
            pltpu.VMEM((C, 64), F32),
            pltpu.VMEM_SHARED((NPAD, 64), F32),
            pltpu.VMEM_SHARED((4 * NPAD,), F32),
            pltpu.SemaphoreType.DMA,
            pltpu.SemaphoreType.DMA,
            pltpu.SemaphoreType.DMA,
            pltpu.SemaphoreType.DMA,
            pltpu.SemaphoreType.DMA,
            pltpu.SemaphoreType.DMA,
            pltpu.SemaphoreType.DMA,
            pltpu.SemaphoreType.DMA,
        ],
    )
    def k(hf0_hbm, hf1_hbm, hf2_hbm, avt_hbm, srcr_hbm, dstr_hbm,
          zacc_hbm, zs_hbm,
          acc_out, s_out,
          asrc, adst, srcv, dstv, sidx, eebuf, rows0, rows1, rows2, rows3,
          acc_sh, s_sh, gsem0, gsem1, gsem2, gsem3,
          ssem0, ssem1, ssem2, ssem3):
        cid = lax.axis_index("c")
        sid = lax.axis_index("s")
        hf_hbm = (hf0_hbm, hf1_hbm, hf2_hbm)
        rowbase = jnp.where(cid == 0, sid * NCH0, 16 * NCH0 + sid * NCH1)
        npairs = jnp.where(cid == 0, NCH0 // 2, NCH1 // 2)

        @pl.when(cid == 0)
        def _():
            pltpu.sync_copy(srcr_hbm.at[pl.ds(rowbase, NCH0)],
                            srcv.at[pl.ds(0, NCH0)])
            pltpu.sync_copy(dstr_hbm.at[pl.ds(rowbase, NCH0)],
                            dstv.at[pl.ds(0, NCH0)])

        @pl.when(cid == 1)
        def _():
            pltpu.sync_copy(srcr_hbm.at[pl.ds(rowbase, NCH1)],
                            srcv.at[pl.ds(0, NCH1)])
            pltpu.sync_copy(dstr_hbm.at[pl.ds(rowbase, NCH1)],
                            dstv.at[pl.ds(0, NCH1)])
        pltpu.sync_copy(zs_hbm.at[pl.ds(sid * _SSL, _SSL)],
                        s_sh.at[pl.ds(sid * _SSL, _SSL)])

        lane = lax.iota(jnp.int32, 16)
        bufs = ((rows0, gsem0, ssem0), (rows1, gsem1, ssem1),
                (rows2, gsem2, ssem2), (rows3, gsem3, ssem3))
        nch = jnp.where(cid == 0, NCH0, NCH1)

        for h in range(NH):
            pltpu.sync_copy(avt_hbm.at[h], asrc)
            pltpu.sync_copy(avt_hbm.at[NH + h], adst)
            pltpu.sync_copy(zacc_hbm.at[pl.ds(sid * _NSL, _NSL)],
                            acc_sh.at[pl.ds(sid * _NSL, _NSL)])
            plsc.subcore_barrier()

            def issue(j, p):
                pltpu.async_copy(hf_hbm[h].at[srcv.at[j]], bufs[p][0],
                                 bufs[p][1])

            def wait_gather(p):
                pltpu.make_async_copy(hf_hbm[h].at[pl.ds(0, C)],
                                      bufs[p][0], bufs[p][1]).wait()

            def issue_scatters(j, p):
                pltpu.async_copy(bufs[p][0], acc_sh.at[dstv.at[j]],
                                 bufs[p][2], add=True)
                pltpu.async_copy(eebuf.at[p], s_sh.at[sidx.at[p]],
                                 bufs[p][2], add=True)

            def wait_scatters(p):
                pltpu.make_async_copy(hf_hbm[h].at[pl.ds(0, C)],
                                      bufs[p][0], bufs[p][2]).wait()
                pltpu.make_async_copy(avt_hbm.at[h, pl.ds(0, C)],
                                      eebuf.at[p], bufs[p][2]).wait()

            def compute_ee(j, p):
                base_g = (rowbase + j) * C
                for k16 in range(C // 16):
                    sv = srcv[j, pl.ds(k16 * 16, 16)]
                    dv = dstv[j, pl.ds(k16 * 16, 16)]
                    valid = (base_g + k16 * 16 + lane) < E
                    _edge_logits(asrc, adst, sv, dv, 0, valid,
                                 eebuf, sidx, k16, h, p)

            def mul(p):
                rows = bufs[p][0]

                def mul_body(g, mc):
                    ee_vec = eebuf[p, pl.ds(g * 16, 16)]
                    for i16 in range(16):
                        i = g * 16 + i16
                        ee_v = jnp.full((16,), ee_vec[i16], F32)
                        for b in range(4):
                            rows[i, pl.ds(b * 16, 16)] = (
                                rows[i, pl.ds(b * 16, 16)] * ee_v)
                    return mc
                lax.fori_loop(0, C // 16, mul_body, 0)

            for p in range(4):
                issue(p, p)

            def quad_body(q, carry):
                for p in range(4):
                    j = q * 4 + p
                    compute_ee(j, p)
                    pprev = (p - 1) % 4

                    @pl.when((j >= 1) & (j + 3 < nch))
                    def _():
                        wait_scatters(pprev)
                        issue(j + 3, pprev)
                    wait_gather(p)
                    mul(p)
                    issue_scatters(j, p)
                return carry
            lax.fori_loop(0, nch // 4, quad_body, 0)
            for p in range(4):
                wait_scatters(p)
            plsc.subcore_barrier()
            pltpu.sync_copy(acc_sh.at[pl.ds(sid * _NSL, _NSL)],
                            acc_out.at[cid, h, pl.ds(sid * _NSL, _NSL)])
            plsc.subcore_barrier()
        pltpu.sync_copy(s_sh.at[pl.ds(sid * _SSL, _SSL)],
                        s_out.at[cid, pl.ds(sid * _SSL, _SSL)])

    return k


def _dot(a, b):
    return lax.dot_general(a, b, (((1,), (0,)), ((), ())),
                           preferred_element_type=F32)


def _dot_rt(a, b):
    return lax.dot_general(a, b, (((1,), (1,)), ((), ())),
                           preferred_element_type=F32)


def _tc1_body(xp, w1cat, a1t, hfeat_out, avt_out):
    h = _dot(xp[...], w1cat[...])
    hfeat_out[...] = h
    avt_out[...] = _dot_rt(a1t[...], h)


_tc1 = pl.pallas_call(
    _tc1_body,
    out_shape=(jax.ShapeDtypeStruct((NPAD, 48), F32),
               jax.ShapeDtypeStruct((8, NPAD), F32)))


def _elu(x):
    return jnp.where(x > 0, x, jnp.exp(jnp.minimum(x, 0.0)) - 1.0)


def _tc2_body(acc1, s1, w2cat, a2t, e1, hf0_out, hf1_out, hf2_out, avt_out):
    accsum = acc1[0] + acc1[1]
    ssum = s1[0] + s1[1]
    rmat = _dot(1.0 / (ssum + 1e-16), e1[...])
    h1 = _elu(accsum * rmat)
    h2f = _dot(h1, w2cat[...])
    hf0_out[...] = h2f[:, 0:64]
    hf1_out[...] = h2f[:, 64:128]
    hf2_out[...] = h2f[:, 128:192]
    avt_out[...] = _dot_rt(a2t[...], h2f)


_tc2 = pl.pallas_call(
    _tc2_body,
    out_shape=(jax.ShapeDtypeStruct((NPAD, 64), F32),
               jax.ShapeDtypeStruct((NPAD, 64), F32),
               jax.ShapeDtypeStruct((NPAD, 64), F32),
               jax.ShapeDtypeStruct((8, NPAD), F32)))


def _tc3_body(acc2, s2, e2, att_w2, wd1, bd1r, wd2, bd2r, out):
    accsum = acc2[0] + acc2[1]
    acat = jnp.concatenate(
        [accsum[0], accsum[1], accsum[2]], axis=1)
    ssum = s2[0] + s2[1]
    rmat = _dot(1.0 / (ssum + 1e-16), e2[...])
    h2 = _elu(acat * rmat)
    th = jnp.tanh(_dot(h2, att_w2[...]))
    ridx = lax.broadcasted_iota(jnp.int32, (NPAD, 1), 0)
    z = jnp.where(ridx < N, th, -1e30)
    p = jnp.exp(z - jnp.max(z))
    scores = p / jnp.sum(p)
    w = h2 * scores
    d1 = jnp.maximum(_dot(w, wd1[...]) + bd1r[...], 0.0)
    out[...] = _dot(d1, wd2[...]) + bd2r[...]


_tc3 = pl.pallas_call(
    _tc3_body,
    out_shape=jax.ShapeDtypeStruct((NPAD, 1), F32),
    compiler_params=pltpu.CompilerParams(
        vmem_limit_bytes=100 * 1024 * 1024))


def kernel(x, edge_index, W1, a_src1, a_dst1, W2, a_src2, a_dst2, att_w,
           Wd1, bd1, Wd2, bd2):
    xp = jnp.pad(x.astype(F32), ((0, NPAD - N), (0, 0)))
    fill = jnp.arange(EPAD - E, dtype=jnp.int32) % N
    src = jnp.concatenate([edge_index[0].astype(jnp.int32), fill])
    dst = jnp.concatenate([edge_index[1].astype(jnp.int32), fill])
    srcp = src.reshape(ROWS_TOT, C)
    dstp = dst.reshape(ROWS_TOT, C)

    w1cat = W1.transpose(1, 0, 2).reshape(11, 48)
    w2cat = W2.transpose(1, 0, 2).reshape(48, 192)
    a1t = jnp.zeros((8, 48), F32)
    a2t = jnp.zeros((8, 192), F32)
    e1 = jnp.zeros((4, 48), F32)
    e2 = jnp.zeros((4, 192), F32)
    for h in range(NH):
        a1t = a1t.at[h, h * 16:(h + 1) * 16].set(a_src1[h])
        a1t = a1t.at[NH + h, h * 16:(h + 1) * 16].set(a_dst1[h])
        a2t = a2t.at[h, h * 64:(h + 1) * 64].set(a_src2[h])
        a2t = a2t.at[NH + h, h * 64:(h + 1) * 64].set(a_dst2[h])
        e1 = e1.at[h, h * 16:(h + 1) * 16].set(1.0)
        e2 = e2.at[h, h * 64:(h + 1) * 64].set(1.0)

    zacc1 = jnp.zeros((NPAD, 48), F32)
    zacc2 = jnp.zeros((NPAD, 64), F32)
    zs = jnp.zeros((4 * NPAD,), F32)

    hfeat1, avt1 = _tc1(xp, w1cat, a1t)
    acc1, s1 = _gat_sc_l1()(hfeat1, avt1, srcp, dstp, zacc1, zs)
    hf0, hf1, hf2, avt2 = _tc2(acc1, s1.reshape(2, NPAD, 4), w2cat, a2t, e1)
    acc2, s2 = _gat_sc_l2()(hf0, hf1, hf2, avt2, srcp, dstp, zacc2, zs)
    o = _tc3(acc2, s2.reshape(2, NPAD, 4), e2, att_w.reshape(192, 1),
             Wd1, bd1.reshape(1, 128), Wd2, bd2.reshape(1, 1))
    return o[:N, 0]


# speedup vs baseline: 2.7360x; 1.0008x over previous
import functools

import jax
import jax.numpy as jnp
from jax import lax
from jax.experimental import pallas as pl
from jax.experimental.pallas import tpu as pltpu
from jax.experimental.pallas import tpu_sc as plsc

N = 10000
NPAD = 10240
E = 320000
NW = 32
C = 128
NCH = 80
TEDGE = NCH * C
EPAD = NW * TEDGE
ROWS_TOT = NW * NCH
NCH0 = 80
NCH1 = 80
NCHMAX = max(NCH0, NCH1)
NH = 3
F32 = jnp.float32

_SC_PARAMS = pltpu.CompilerParams(
    needs_layout_passes=False, use_tc_tiling_on_sc=False)
_MESH = dict(core_axis_name="c", subcore_axis_name="s", num_cores=2,
             num_subcores=16)
_NSL = NPAD // 16
_SSL = 4 * NPAD // 16


def _edge_logits(asrc, adst, sv, dv, off, valid, eebuf, sidx, k16, h, hs):
    a = plsc.load_gather(asrc, [sv + off])
    b = plsc.load_gather(adst, [dv + off])
    e = a + b
    e = jnp.maximum(e, 0.2 * e)
    ee = jnp.where(valid, jnp.exp(e), 0.0)
    eebuf[hs, pl.ds(k16 * 16, 16)] = ee
    sidx[hs, pl.ds(k16 * 16, 16)] = dv * 4 + h


@functools.lru_cache(maxsize=None)
def _gat_sc_l1():
    @functools.partial(
        pl.kernel,
        out_type=(
            jax.ShapeDtypeStruct((2, NPAD, 48), F32),
            jax.ShapeDtypeStruct((2, 4 * NPAD), F32),
        ),
        mesh=plsc.VectorSubcoreMesh(**_MESH),
        compiler_params=_SC_PARAMS,
        scratch_types=[
            pltpu.VMEM((NH * NPAD,), F32),
            pltpu.VMEM((NH * NPAD,), F32),
            pltpu.VMEM((NCH, C), jnp.int32),
            pltpu.VMEM((NCH, C), jnp.int32),
            pltpu.VMEM((2 * NH, C), jnp.int32),
            pltpu.VMEM((2 * NH, C), F32),
            pltpu.VMEM((C, 48), F32),
            pltpu.VMEM((C, 48), F32),
            pltpu.VMEM_SHARED((NPAD, 48), F32),
            pltpu.VMEM_SHARED((4 * NPAD,), F32),
            pltpu.SemaphoreType.DMA,
            pltpu.SemaphoreType.DMA,
            pltpu.SemaphoreType.DMA,
            pltpu.SemaphoreType.DMA,
        ],
    )
    def k(hfeat_hbm, avt_hbm, srcr_hbm, dstr_hbm, zacc_hbm, zs_hbm,
          acc_out, s_out,
          asrc, adst, srcv, dstv, sidx, eebuf, rows0, rows1,
          acc_sh, s_sh, gsem0, gsem1, ssem0, ssem1):
        cid = lax.axis_index("c")
        sid = lax.axis_index("s")
        wid = sid * 2 + cid
        for h in range(NH):
            pltpu.sync_copy(avt_hbm.at[h], asrc.at[pl.ds(h * NPAD, NPAD)])
            pltpu.sync_copy(avt_hbm.at[NH + h],
                            adst.at[pl.ds(h * NPAD, NPAD)])
        pltpu.sync_copy(srcr_hbm.at[pl.ds(wid * NCH, NCH)], srcv)
        pltpu.sync_copy(dstr_hbm.at[pl.ds(wid * NCH, NCH)], dstv)
        pltpu.sync_copy(zacc_hbm.at[pl.ds(sid * _NSL, _NSL)],
                        acc_sh.at[pl.ds(sid * _NSL, _NSL)])
        pltpu.sync_copy(zs_hbm.at[pl.ds(sid * _SSL, _SSL)],
                        s_sh.at[pl.ds(sid * _SSL, _SSL)])
        plsc.subcore_barrier()

        lane = lax.iota(jnp.int32, 16)
        bufs = ((rows0, gsem0, ssem0), (rows1, gsem1, ssem1))

        def issue(j, p):
            pltpu.async_copy(hfeat_hbm.at[srcv.at[j]], bufs[p][0],
                             bufs[p][1])

        def wait_gather(p):
            pltpu.make_async_copy(hfeat_hbm.at[pl.ds(0, C)], bufs[p][0],
                                  bufs[p][1]).wait()

        def issue_scatters(j, p):
            pltpu.async_copy(bufs[p][0], acc_sh.at[dstv.at[j]],
                             bufs[p][2], add=True)
            for h in range(NH):
                pltpu.async_copy(eebuf.at[p * NH + h],
                                 s_sh.at[sidx.at[p * NH + h]],
                                 bufs[p][2], add=True)

        def wait_scatters(p):
            pltpu.make_async_copy(hfeat_hbm.at[pl.ds(0, C)], bufs[p][0],
                                  bufs[p][2]).wait()
            for h in range(NH):
                pltpu.make_async_copy(avt_hbm.at[h, pl.ds(0, C)],
                                      eebuf.at[p * NH + h],
                                      bufs[p][2]).wait()

        def compute_ee(j, p):
            base_g = (wid * NCH + j) * C
            for k16 in range(C // 16):
                sv = srcv[j, pl.ds(k16 * 16, 16)]
                dv = dstv[j, pl.ds(k16 * 16, 16)]
                valid = (base_g + k16 * 16 + lane) < E
                for h in range(NH):
                    _edge_logits(asrc, adst, sv, dv, h * NPAD, valid,
                                 eebuf, sidx, k16, h, p * NH + h)

        def mul(p):
            rows = bufs[p][0]

            def mul_body(g, mc):
                ee_vecs = [eebuf[p * NH + h, pl.ds(g * 16, 16)]
                           for h in range(NH)]
                for i16 in range(16):
                    i = g * 16 + i16
                    for h in range(NH):
                        rows[i, pl.ds(h * 16, 16)] = (
                            rows[i, pl.ds(h * 16, 16)] * ee_vecs[h][i16])
                return mc
            lax.fori_loop(0, C // 16, mul_body, 0)

        issue(0, 0)
        issue(1, 1)

        def pair_body(j2, carry):
            j0 = j2 * 2
            compute_ee(j0, 0)

            @pl.when(j2 > 0)
            def _():
                wait_scatters(1)
                issue(j0 + 1, 1)
            wait_gather(0)
            mul(0)
            issue_scatters(j0, 0)
            compute_ee(j0 + 1, 1)
            wait_scatters(0)

            @pl.when(j0 + 2 < NCH)
            def _():
                issue(j0 + 2, 0)
            wait_gather(1)
            mul(1)
            issue_scatters(j0 + 1, 1)
            return carry
        lax.fori_loop(0, NCH // 2, pair_body, 0)
        wait_scatters(1)
        plsc.subcore_barrier()
        pltpu.sync_copy(acc_sh.at[pl.ds(sid * _NSL, _NSL)],
                        acc_out.at[cid, pl.ds(sid * _NSL, _NSL)])
        pltpu.sync_copy(s_sh.at[pl.ds(sid * _SSL, _SSL)],
                        s_out.at[cid, pl.ds(sid * _SSL, _SSL)])

    return k


@functools.lru_cache(maxsize=None)
def _gat_sc_l2():
    @functools.partial(
        pl.kernel,
        out_type=(
            jax.ShapeDtypeStruct((2, NH, NPAD, 64), F32),
            jax.ShapeDtypeStruct((2, 4 * NPAD), F32),
        ),
        mesh=plsc.VectorSubcoreMesh(**_MESH),
        compiler_params=_SC_PARAMS,
        scratch_types=[
            pltpu.VMEM((NPAD,), F32),
            pltpu.VMEM((NPAD,), F32),
            pltpu.VMEM((NCHMAX, C), jnp.int32),
            pltpu.VMEM((NCHMAX, C), jnp.int32),
            pltpu.VMEM((4, C), jnp.int32),
            pltpu.VMEM((4, C), F32),
            pltpu.VMEM((C, 64), F32),
            pltpu.VMEM((C, 64), F32),
            pltpu.VMEM((C, 64), F32),
            pltpu.VMEM((C, 64), F32),
            pltpu.VMEM_SHARED((NPAD, 64), F32),
            pltpu.VMEM_SHARED((4 * NPAD,), F32),
            pltpu.SemaphoreType.DMA,
            pltpu.SemaphoreType.DMA,
            pltpu.SemaphoreType.DMA,
            pltpu.SemaphoreType.DMA,
            pltpu.SemaphoreType.DMA,
            pltpu.SemaphoreType.DMA,
            pltpu.SemaphoreType.DMA,
            pltpu.SemaphoreType.DMA,
        ],
    )
    def k(hf0_hbm, hf1_hbm, hf2_hbm, avt_hbm, srcr_hbm, dstr_hbm,
          zacc_hbm, zs_hbm,
          acc_out, s_out,
          asrc, adst, srcv, dstv, sidx, eebuf, rows0, rows1, rows2, rows3,
          acc_sh, s_sh, gsem0, gsem1, gsem2, gsem3,
          ssem0, ssem1, ssem2, ssem3):
        cid = lax.axis_index("c")
        sid = lax.axis_index("s")
        hf_hbm = (hf0_hbm, hf1_hbm, hf2_hbm)
        rowbase = jnp.where(cid == 0, sid * NCH0, 16 * NCH0 + sid * NCH1)
        npairs = jnp.where(cid == 0, NCH0 // 2, NCH1 // 2)

        @pl.when(cid == 0)
        def _():
            pltpu.sync_copy(srcr_hbm.at[pl.ds(rowbase, NCH0)],
                            srcv.at[pl.ds(0, NCH0)])
            pltpu.sync_copy(dstr_hbm.at[pl.ds(rowbase, NCH0)],
                            dstv.at[pl.ds(0, NCH0)])

        @pl.when(cid == 1)
        def _():
            pltpu.sync_copy(srcr_hbm.at[pl.ds(rowbase, NCH1)],
                            srcv.at[pl.ds(0, NCH1)])
            pltpu.sync_copy(dstr_hbm.at[pl.ds(rowbase, NCH1)],
                            dstv.at[pl.ds(0, NCH1)])
        pltpu.sync_copy(zs_hbm.at[pl.ds(sid * _SSL, _SSL)],
                        s_sh.at[pl.ds(sid * _SSL, _SSL)])

        lane = lax.iota(jnp.int32, 16)
        bufs = ((rows0, gsem0, ssem0), (rows1, gsem1, ssem1),
                (rows2, gsem2, ssem2), (rows3, gsem3, ssem3))
        nch = jnp.where(cid == 0, NCH0, NCH1)

        for h in range(NH):
            pltpu.sync_copy(avt_hbm.at[h], asrc)
            pltpu.sync_copy(avt_hbm.at[NH + h], adst)
            pltpu.sync_copy(zacc_hbm.at[pl.ds(sid * _NSL, _NSL)],
                            acc_sh.at[pl.ds(sid * _NSL, _NSL)])
            plsc.subcore_barrier()

            def issue(j, p):
                pltpu.async_copy(hf_hbm[h].at[srcv.at[j]], bufs[p][0],
                                 bufs[p][1])

            def wait_gather(p):
                pltpu.make_async_copy(hf_hbm[h].at[pl.ds(0, C)],
                                      bufs[p][0], bufs[p][1]).wait()

            def issue_scatters(j, p):
                pltpu.async_copy(bufs[p][0], acc_sh.at[dstv.at[j]],
                                 bufs[p][2], add=True)
                pltpu.async_copy(eebuf.at[p], s_sh.at[sidx.at[p]],
                                 bufs[p][2], add=True)

            def wait_scatters(p):
                pltpu.make_async_copy(hf_hbm[h].at[pl.ds(0, C)],
                                      bufs[p][0], bufs[p][2]).wait()
                pltpu.make_async_copy(avt_hbm.at[h, pl.ds(0, C)],
                                      eebuf.at[p], bufs[p][2]).wait()

            def compute_ee(j, p):
                base_g = (rowbase + j) * C
                for k16 in range(C // 16):
                    sv = srcv[j, pl.ds(k16 * 16, 16)]
                    dv = dstv[j, pl.ds(k16 * 16, 16)]
                    valid = (base_g + k16 * 16 + lane) < E
                    _edge_logits(asrc, adst, sv, dv, 0, valid,
                                 eebuf, sidx, k16, h, p)

            def mul(p):
                rows = bufs[p][0]

                def mul_body(g, mc):
                    ee_vec = eebuf[p, pl.ds(g * 16, 16)]
                    for i16 in range(16):
                        i = g * 16 + i16
                        ee_s = ee_vec[i16]
                        for b in range(4):
                            rows[i, pl.ds(b * 16, 16)] = (
                                rows[i, pl.ds(b * 16, 16)] * ee_s)
                    return mc
                lax.fori_loop(0, C // 16, mul_body, 0)

            for p in range(4):
                issue(p, p)

            def quad_body(q, carry):
                for p in range(4):
                    j = q * 4 + p
                    compute_ee(j, p)
                    pprev = (p - 1) % 4

                    @pl.when((j >= 1) & (j + 3 < nch))
                    def _():
                        wait_scatters(pprev)
                        issue(j + 3, pprev)
                    wait_gather(p)
                    mul(p)
                    issue_scatters(j, p)
                return carry
            lax.fori_loop(0, nch // 4, quad_body, 0)
            for p in range(4):
                wait_scatters(p)
            plsc.subcore_barrier()
            pltpu.sync_copy(acc_sh.at[pl.ds(sid * _NSL, _NSL)],
                            acc_out.at[cid, h, pl.ds(sid * _NSL, _NSL)])
            plsc.subcore_barrier()
        pltpu.sync_copy(s_sh.at[pl.ds(sid * _SSL, _SSL)],
                        s_out.at[cid, pl.ds(sid * _SSL, _SSL)])

    return k


def _dot(a, b):
    return lax.dot_general(a, b, (((1,), (0,)), ((), ())),
                           preferred_element_type=F32)


def _dot_rt(a, b):
    return lax.dot_general(a, b, (((1,), (1,)), ((), ())),
                           preferred_element_type=F32)


def _tc1_body(xp, w1cat, a1t, hfeat_out, avt_out):
    h = _dot(xp[...], w1cat[...])
    hfeat_out[...] = h
    avt_out[...] = _dot_rt(a1t[...], h)


_tc1 = pl.pallas_call(
    _tc1_body,
    out_shape=(jax.ShapeDtypeStruct((NPAD, 48), F32),
               jax.ShapeDtypeStruct((8, NPAD), F32)))


def _elu(x):
    return jnp.where(x > 0, x, jnp.exp(jnp.minimum(x, 0.0)) - 1.0)


def _tc2_body(acc1, s1, w2cat, a2t, e1, hf0_out, hf1_out, hf2_out, avt_out):
    accsum = acc1[0] + acc1[1]
    ssum = s1[0] + s1[1]
    rmat = _dot(1.0 / (ssum + 1e-16), e1[...])
    h1 = _elu(accsum * rmat)
    h2f = _dot(h1, w2cat[...])
    hf0_out[...] = h2f[:, 0:64]
    hf1_out[...] = h2f[:, 64:128]
    hf2_out[...] = h2f[:, 128:192]
    avt_out[...] = _dot_rt(a2t[...], h2f)


_tc2 = pl.pallas_call(
    _tc2_body,
    out_shape=(jax.ShapeDtypeStruct((NPAD, 64), F32),
               jax.ShapeDtypeStruct((NPAD, 64), F32),
               jax.ShapeDtypeStruct((NPAD, 64), F32),
               jax.ShapeDtypeStruct((8, NPAD), F32)))


def _tc3_body(acc2, s2, e2, att_w2, wd1, bd1r, wd2, bd2r, out):
    accsum = acc2[0] + acc2[1]
    acat = jnp.concatenate(
        [accsum[0], accsum[1], accsum[2]], axis=1)
    ssum = s2[0] + s2[1]
    rmat = _dot(1.0 / (ssum + 1e-16), e2[...])
    h2 = _elu(acat * rmat)
    th = jnp.tanh(_dot(h2, att_w2[...]))
    ridx = lax.broadcasted_iota(jnp.int32, (NPAD, 1), 0)
    z = jnp.where(ridx < N, th, -1e30)
    p = jnp.exp(z - jnp.max(z))
    scores = p / jnp.sum(p)
    w = h2 * scores
    d1 = jnp.maximum(_dot(w, wd1[...]) + bd1r[...], 0.0)
    out[...] = _dot(d1, wd2[...]) + bd2r[...]


_tc3 = pl.pallas_call(
    _tc3_body,
    out_shape=jax.ShapeDtypeStruct((NPAD, 1), F32),
    compiler_params=pltpu.CompilerParams(
        vmem_limit_bytes=100 * 1024 * 1024))


def kernel(x, edge_index, W1, a_src1, a_dst1, W2, a_src2, a_dst2, att_w,
           Wd1, bd1, Wd2, bd2):
    xp = jnp.pad(x.astype(F32), ((0, NPAD - N), (0, 0)))
    fill = jnp.arange(EPAD - E, dtype=jnp.int32) % N
    src = jnp.concatenate([edge_index[0].astype(jnp.int32), fill])
    dst = jnp.concatenate([edge_index[1].astype(jnp.int32), fill])
    srcp = src.reshape(ROWS_TOT, C)
    dstp = dst.reshape(ROWS_TOT, C)

    w1cat = W1.transpose(1, 0, 2).reshape(11, 48)
    w2cat = W2.transpose(1, 0, 2).reshape(48, 192)
    a1t = jnp.zeros((8, 48), F32)
    a2t = jnp.zeros((8, 192), F32)
    e1 = jnp.zeros((4, 48), F32)
    e2 = jnp.zeros((4, 192), F32)
    for h in range(NH):
        a1t = a1t.at[h, h * 16:(h + 1) * 16].set(a_src1[h])
        a1t = a1t.at[NH + h, h * 16:(h + 1) * 16].set(a_dst1[h])
        a2t = a2t.at[h, h * 64:(h + 1) * 64].set(a_src2[h])
        a2t = a2t.at[NH + h, h * 64:(h + 1) * 64].set(a_dst2[h])
        e1 = e1.at[h, h * 16:(h + 1) * 16].set(1.0)
        e2 = e2.at[h, h * 64:(h + 1) * 64].set(1.0)

    zacc1 = jnp.zeros((NPAD, 48), F32)
    zacc2 = jnp.zeros((NPAD, 64), F32)
    zs = jnp.zeros((4 * NPAD,), F32)

    hfeat1, avt1 = _tc1(xp, w1cat, a1t)
    acc1, s1 = _gat_sc_l1()(hfeat1, avt1, srcp, dstp, zacc1, zs)
    hf0, hf1, hf2, avt2 = _tc2(acc1, s1.reshape(2, NPAD, 4), w2cat, a2t, e1)
    acc2, s2 = _gat_sc_l2()(hf0, hf1, hf2, avt2, srcp, dstp, zacc2, zs)
    o = _tc3(acc2, s2.reshape(2, NPAD, 4), e2, att_w.reshape(192, 1),
             Wd1, bd1.reshape(1, 128), Wd2, bd2.reshape(1, 1))
    return o[:N, 0]


# speedup vs baseline: 2.7428x; 1.0025x over previous
import functools

import jax
import jax.numpy as jnp
from jax import lax
from jax.experimental import pallas as pl
from jax.experimental.pallas import tpu as pltpu
from jax.experimental.pallas import tpu_sc as plsc

N = 10000
NPAD = 10240
E = 320000
NW = 32
C = 128
NCH = 80
TEDGE = NCH * C
EPAD = NW * TEDGE
ROWS_TOT = NW * NCH
NCH0 = 80
NCH1 = 80
NCHMAX = max(NCH0, NCH1)
NH = 3
F32 = jnp.float32

_SC_PARAMS = pltpu.CompilerParams(
    needs_layout_passes=False, use_tc_tiling_on_sc=False)
_MESH = dict(core_axis_name="c", subcore_axis_name="s", num_cores=2,
             num_subcores=16)
_NSL = NPAD // 16
_SSL = 4 * NPAD // 16


def _edge_logits(asrc, adst, sv, dv, off, eebuf, sidx, k16, h, hs):
    a = plsc.load_gather(asrc, [sv + off])
    b = plsc.load_gather(adst, [dv + off])
    e = a + b
    e = jnp.maximum(e, 0.2 * e)
    ee = jnp.exp(e)
    eebuf[hs, pl.ds(k16 * 16, 16)] = ee
    sidx[hs, pl.ds(k16 * 16, 16)] = dv * 4 + h


@functools.lru_cache(maxsize=None)
def _gat_sc_l1():
    @functools.partial(
        pl.kernel,
        out_type=(
            jax.ShapeDtypeStruct((2, NPAD, 48), F32),
            jax.ShapeDtypeStruct((2, 4 * NPAD), F32),
        ),
        mesh=plsc.VectorSubcoreMesh(**_MESH),
        compiler_params=_SC_PARAMS,
        scratch_types=[
            pltpu.VMEM((NH * NPAD,), F32),
            pltpu.VMEM((NH * NPAD,), F32),
            pltpu.VMEM((NCH, C), jnp.int32),
            pltpu.VMEM((NCH, C), jnp.int32),
            pltpu.VMEM((2 * NH, C), jnp.int32),
            pltpu.VMEM((2 * NH, C), F32),
            pltpu.VMEM((C, 48), F32),
            pltpu.VMEM((C, 48), F32),
            pltpu.VMEM_SHARED((NPAD, 48), F32),
            pltpu.VMEM_SHARED((4 * NPAD,), F32),
            pltpu.SemaphoreType.DMA,
            pltpu.SemaphoreType.DMA,
            pltpu.SemaphoreType.DMA,
            pltpu.SemaphoreType.DMA,
        ],
    )
    def k(hfeat_hbm, avt_hbm, srcr_hbm, dstr_hbm, zacc_hbm, zs_hbm,
          acc_out, s_out,
          asrc, adst, srcv, dstv, sidx, eebuf, rows0, rows1,
          acc_sh, s_sh, gsem0, gsem1, ssem0, ssem1):
        cid = lax.axis_index("c")
        sid = lax.axis_index("s")
        wid = sid * 2 + cid
        for h in range(NH):
            pltpu.sync_copy(avt_hbm.at[h], asrc.at[pl.ds(h * NPAD, NPAD)])
            pltpu.sync_copy(avt_hbm.at[NH + h],
                            adst.at[pl.ds(h * NPAD, NPAD)])
        pltpu.sync_copy(srcr_hbm.at[pl.ds(wid * NCH, NCH)], srcv)
        pltpu.sync_copy(dstr_hbm.at[pl.ds(wid * NCH, NCH)], dstv)
        pltpu.sync_copy(zacc_hbm.at[pl.ds(sid * _NSL, _NSL)],
                        acc_sh.at[pl.ds(sid * _NSL, _NSL)])
        pltpu.sync_copy(zs_hbm.at[pl.ds(sid * _SSL, _SSL)],
                        s_sh.at[pl.ds(sid * _SSL, _SSL)])
        plsc.subcore_barrier()

        bufs = ((rows0, gsem0, ssem0), (rows1, gsem1, ssem1))

        def issue(j, p):
            pltpu.async_copy(hfeat_hbm.at[srcv.at[j]], bufs[p][0],
                             bufs[p][1])

        def wait_gather(p):
            pltpu.make_async_copy(hfeat_hbm.at[pl.ds(0, C)], bufs[p][0],
                                  bufs[p][1]).wait()

        def issue_scatters(j, p):
            pltpu.async_copy(bufs[p][0], acc_sh.at[dstv.at[j]],
                             bufs[p][2], add=True)
            for h in range(NH):
                pltpu.async_copy(eebuf.at[p * NH + h],
                                 s_sh.at[sidx.at[p * NH + h]],
                                 bufs[p][2], add=True)

        def wait_scatters(p):
            pltpu.make_async_copy(hfeat_hbm.at[pl.ds(0, C)], bufs[p][0],
                                  bufs[p][2]).wait()
            for h in range(NH):
                pltpu.make_async_copy(avt_hbm.at[h, pl.ds(0, C)],
                                      eebuf.at[p * NH + h],
                                      bufs[p][2]).wait()

        def compute_ee(j, p):
            for k16 in range(C // 16):
                sv = srcv[j, pl.ds(k16 * 16, 16)]
                dv = dstv[j, pl.ds(k16 * 16, 16)]
                for h in range(NH):
                    _edge_logits(asrc, adst, sv, dv, h * NPAD,
                                 eebuf, sidx, k16, h, p * NH + h)

        def mul(p):
            rows = bufs[p][0]

            def mul_body(g, mc):
                ee_vecs = [eebuf[p * NH + h, pl.ds(g * 16, 16)]
                           for h in range(NH)]
                for i16 in range(16):
                    i = g * 16 + i16
                    for h in range(NH):
                        rows[i, pl.ds(h * 16, 16)] = (
                            rows[i, pl.ds(h * 16, 16)] * ee_vecs[h][i16])
                return mc
            lax.fori_loop(0, C // 16, mul_body, 0)

        issue(0, 0)
        issue(1, 1)

        def pair_body(j2, carry):
            j0 = j2 * 2
            compute_ee(j0, 0)

            @pl.when(j2 > 0)
            def _():
                wait_scatters(1)
                issue(j0 + 1, 1)
            wait_gather(0)
            mul(0)
            issue_scatters(j0, 0)
            compute_ee(j0 + 1, 1)
            wait_scatters(0)

            @pl.when(j0 + 2 < NCH)
            def _():
                issue(j0 + 2, 0)
            wait_gather(1)
            mul(1)
            issue_scatters(j0 + 1, 1)
            return carry
        lax.fori_loop(0, NCH // 2, pair_body, 0)
        wait_scatters(1)
        plsc.subcore_barrier()
        pltpu.sync_copy(acc_sh.at[pl.ds(sid * _NSL, _NSL)],
                        acc_out.at[cid, pl.ds(sid * _NSL, _NSL)])
        pltpu.sync_copy(s_sh.at[pl.ds(sid * _SSL, _SSL)],
                        s_out.at[cid, pl.ds(sid * _SSL, _SSL)])

    return k


@functools.lru_cache(maxsize=None)
def _gat_sc_l2():
    @functools.partial(
        pl.kernel,
        out_type=(
            jax.ShapeDtypeStruct((2, NH, NPAD, 64), F32),
            jax.ShapeDtypeStruct((2, 4 * NPAD), F32),
        ),
        mesh=plsc.VectorSubcoreMesh(**_MESH),
        compiler_params=_SC_PARAMS,
        scratch_types=[
            pltpu.VMEM((NPAD,), F32),
            pltpu.VMEM((NPAD,), F32),
            pltpu.VMEM((NCHMAX, C), jnp.int32),
            pltpu.VMEM((NCHMAX, C), jnp.int32),
            pltpu.VMEM((4, C), jnp.int32),
            pltpu.VMEM((4, C), F32),
            pltpu.VMEM((C, 64), F32),
            pltpu.VMEM((C, 64), F32),
            pltpu.VMEM((C, 64), F32),
            pltpu.VMEM((C, 64), F32),
            pltpu.VMEM_SHARED((NPAD, 64), F32),
            pltpu.VMEM_SHARED((4 * NPAD,), F32),
            pltpu.SemaphoreType.DMA,
            pltpu.SemaphoreType.DMA,
            pltpu.SemaphoreType.DMA,
            pltpu.SemaphoreType.DMA,
            pltpu.SemaphoreType.DMA,
            pltpu.SemaphoreType.DMA,
            pltpu.SemaphoreType.DMA,
            pltpu.SemaphoreType.DMA,
        ],
    )
    def k(hf0_hbm, hf1_hbm, hf2_hbm, avt_hbm, srcr_hbm, dstr_hbm,
          zacc_hbm, zs_hbm,
          acc_out, s_out,
          asrc, adst, srcv, dstv, sidx, eebuf, rows0, rows1, rows2, rows3,
          acc_sh, s_sh, gsem0, gsem1, gsem2, gsem3,
          ssem0, ssem1, ssem2, ssem3):
        cid = lax.axis_index("c")
        sid = lax.axis_index("s")
        hf_hbm = (hf0_hbm, hf1_hbm, hf2_hbm)
        rowbase = jnp.where(cid == 0, sid * NCH0, 16 * NCH0 + sid * NCH1)
        npairs = jnp.where(cid == 0, NCH0 // 2, NCH1 // 2)

        @pl.when(cid == 0)
        def _():
            pltpu.sync_copy(srcr_hbm.at[pl.ds(rowbase, NCH0)],
                            srcv.at[pl.ds(0, NCH0)])
            pltpu.sync_copy(dstr_hbm.at[pl.ds(rowbase, NCH0)],
                            dstv.at[pl.ds(0, NCH0)])

        @pl.when(cid == 1)
        def _():
            pltpu.sync_copy(srcr_hbm.at[pl.ds(rowbase, NCH1)],
                            srcv.at[pl.ds(0, NCH1)])
            pltpu.sync_copy(dstr_hbm.at[pl.ds(rowbase, NCH1)],
                            dstv.at[pl.ds(0, NCH1)])
        pltpu.sync_copy(zs_hbm.at[pl.ds(sid * _SSL, _SSL)],
                        s_sh.at[pl.ds(sid * _SSL, _SSL)])

        bufs = ((rows0, gsem0, ssem0), (rows1, gsem1, ssem1),
                (rows2, gsem2, ssem2), (rows3, gsem3, ssem3))
        nch = jnp.where(cid == 0, NCH0, NCH1)

        for h in range(NH):
            pltpu.sync_copy(avt_hbm.at[h], asrc)
            pltpu.sync_copy(avt_hbm.at[NH + h], adst)
            pltpu.sync_copy(zacc_hbm.at[pl.ds(sid * _NSL, _NSL)],
                            acc_sh.at[pl.ds(sid * _NSL, _NSL)])
            plsc.subcore_barrier()

            def issue(j, p):
                pltpu.async_copy(hf_hbm[h].at[srcv.at[j]], bufs[p][0],
                                 bufs[p][1])

            def wait_gather(p):
                pltpu.make_async_copy(hf_hbm[h].at[pl.ds(0, C)],
                                      bufs[p][0], bufs[p][1]).wait()

            def issue_scatters(j, p):
                pltpu.async_copy(bufs[p][0], acc_sh.at[dstv.at[j]],
                                 bufs[p][2], add=True)
                pltpu.async_copy(eebuf.at[p], s_sh.at[sidx.at[p]],
                                 bufs[p][2], add=True)

            def wait_scatters(p):
                pltpu.make_async_copy(hf_hbm[h].at[pl.ds(0, C)],
                                      bufs[p][0], bufs[p][2]).wait()
                pltpu.make_async_copy(avt_hbm.at[h, pl.ds(0, C)],
                                      eebuf.at[p], bufs[p][2]).wait()

            def compute_ee(j, p):
                for k16 in range(C // 16):
                    sv = srcv[j, pl.ds(k16 * 16, 16)]
                    dv = dstv[j, pl.ds(k16 * 16, 16)]
                    _edge_logits(asrc, adst, sv, dv, 0,
                                 eebuf, sidx, k16, h, p)

            def mul(p):
                rows = bufs[p][0]

                def mul_body(g, mc):
                    ee_vec = eebuf[p, pl.ds(g * 16, 16)]
                    for i16 in range(16):
                        i = g * 16 + i16
                        ee_s = ee_vec[i16]
                        for b in range(4):
                            rows[i, pl.ds(b * 16, 16)] = (
                                rows[i, pl.ds(b * 16, 16)] * ee_s)
                    return mc
                lax.fori_loop(0, C // 16, mul_body, 0)

            for p in range(4):
                issue(p, p)

            def quad_body(q, carry):
                for p in range(4):
                    j = q * 4 + p
                    compute_ee(j, p)
                    pprev = (p - 1) % 4

                    @pl.when((j >= 1) & (j + 3 < nch))
                    def _():
                        wait_scatters(pprev)
                        issue(j + 3, pprev)
                    wait_gather(p)
                    mul(p)
                    issue_scatters(j, p)
                return carry
            lax.fori_loop(0, nch // 4, quad_body, 0)
            for p in range(4):
                wait_scatters(p)
            plsc.subcore_barrier()
            pltpu.sync_copy(acc_sh.at[pl.ds(sid * _NSL, _NSL)],
                            acc_out.at[cid, h, pl.ds(sid * _NSL, _NSL)])
            plsc.subcore_barrier()
        pltpu.sync_copy(s_sh.at[pl.ds(sid * _SSL, _SSL)],
                        s_out.at[cid, pl.ds(sid * _SSL, _SSL)])

    return k


def _dot(a, b):
    return lax.dot_general(a, b, (((1,), (0,)), ((), ())),
                           preferred_element_type=F32)


def _dot_rt(a, b):
    return lax.dot_general(a, b, (((1,), (1,)), ((), ())),
                           preferred_element_type=F32)


def _tc1_body(xp, w1cat, a1t, hfeat_out, avt_out):
    h = _dot(xp[...], w1cat[...])
    hfeat_out[...] = h
    avt_out[...] = _dot_rt(a1t[...], h)


_tc1 = pl.pallas_call(
    _tc1_body,
    out_shape=(jax.ShapeDtypeStruct((NPAD, 48), F32),
               jax.ShapeDtypeStruct((8, NPAD), F32)))


def _elu(x):
    return jnp.where(x > 0, x, jnp.exp(jnp.minimum(x, 0.0)) - 1.0)


def _tc2_body(acc1, s1, w2cat, a2t, e1, hf0_out, hf1_out, hf2_out, avt_out):
    accsum = acc1[0] + acc1[1]
    ssum = s1[0] + s1[1]
    rmat = _dot(1.0 / (ssum + 1e-16), e1[...])
    h1 = _elu(accsum * rmat)
    h2f = _dot(h1, w2cat[...])
    hf0_out[...] = h2f[:, 0:64]
    hf1_out[...] = h2f[:, 64:128]
    hf2_out[...] = h2f[:, 128:192]
    avt_out[...] = _dot_rt(a2t[...], h2f)


_tc2 = pl.pallas_call(
    _tc2_body,
    out_shape=(jax.ShapeDtypeStruct((NPAD, 64), F32),
               jax.ShapeDtypeStruct((NPAD, 64), F32),
               jax.ShapeDtypeStruct((NPAD, 64), F32),
               jax.ShapeDtypeStruct((8, NPAD), F32)))


def _tc3_body(acc2, s2, e2, att_w2, wd1, bd1r, wd2, bd2r, out):
    accsum = acc2[0] + acc2[1]
    acat = jnp.concatenate(
        [accsum[0], accsum[1], accsum[2]], axis=1)
    ssum = s2[0] + s2[1]
    rmat = _dot(1.0 / (ssum + 1e-16), e2[...])
    h2 = _elu(acat * rmat)
    th = jnp.tanh(_dot(h2, att_w2[...]))
    ridx = lax.broadcasted_iota(jnp.int32, (NPAD, 1), 0)
    z = jnp.where(ridx < N, th, -1e30)
    p = jnp.exp(z - jnp.max(z))
    scores = p / jnp.sum(p)
    w = h2 * scores
    d1 = jnp.maximum(_dot(w, wd1[...]) + bd1r[...], 0.0)
    out[...] = _dot(d1, wd2[...]) + bd2r[...]


_tc3 = pl.pallas_call(
    _tc3_body,
    out_shape=jax.ShapeDtypeStruct((NPAD, 1), F32),
    compiler_params=pltpu.CompilerParams(
        vmem_limit_bytes=100 * 1024 * 1024))


def kernel(x, edge_index, W1, a_src1, a_dst1, W2, a_src2, a_dst2, att_w,
           Wd1, bd1, Wd2, bd2):
    xp = jnp.pad(x.astype(F32), ((0, NPAD - N), (0, 0)))
    fill = N + jnp.arange(EPAD - E, dtype=jnp.int32) % (NPAD - N)
    src = jnp.concatenate([edge_index[0].astype(jnp.int32), fill])
    dst = jnp.concatenate([edge_index[1].astype(jnp.int32), fill])
    srcp = src.reshape(ROWS_TOT, C)
    dstp = dst.reshape(ROWS_TOT, C)

    w1cat = W1.transpose(1, 0, 2).reshape(11, 48)
    w2cat = W2.transpose(1, 0, 2).reshape(48, 192)
    a1t = jnp.zeros((8, 48), F32)
    a2t = jnp.zeros((8, 192), F32)
    e1 = jnp.zeros((4, 48), F32)
    e2 = jnp.zeros((4, 192), F32)
    for h in range(NH):
        a1t = a1t.at[h, h * 16:(h + 1) * 16].set(a_src1[h])
        a1t = a1t.at[NH + h, h * 16:(h + 1) * 16].set(a_dst1[h])
        a2t = a2t.at[h, h * 64:(h + 1) * 64].set(a_src2[h])
        a2t = a2t.at[NH + h, h * 64:(h + 1) * 64].set(a_dst2[h])
        e1 = e1.at[h, h * 16:(h + 1) * 16].set(1.0)
        e2 = e2.at[h, h * 64:(h + 1) * 64].set(1.0)

    zacc1 = jnp.zeros((NPAD, 48), F32)
    zacc2 = jnp.zeros((NPAD, 64), F32)
    zs = jnp.zeros((4 * NPAD,), F32)

    hfeat1, avt1 = _tc1(xp, w1cat, a1t)
    acc1, s1 = _gat_sc_l1()(hfeat1, avt1, srcp, dstp, zacc1, zs)
    hf0, hf1, hf2, avt2 = _tc2(acc1, s1.reshape(2, NPAD, 4), w2cat, a2t, e1)
    acc2, s2 = _gat_sc_l2()(hf0, hf1, hf2, avt2, srcp, dstp, zacc2, zs)
    o = _tc3(acc2, s2.reshape(2, NPAD, 4), e2, att_w.reshape(192, 1),
             Wd1, bd1.reshape(1, 128), Wd2, bd2.reshape(1, 1))
    return o[:N, 0]
